# baseline (speedup 1.0000x reference)
import functools

import jax
import jax.numpy as jnp
from jax import lax
from jax.experimental import pallas as pl
from jax.experimental.pallas import tpu as pltpu
from jax.experimental.pallas import tpu_sc as plsc

D = 300
B = 1024
L = 200
HID = 32
OUT = 2

NC = 2
NS = 16
NW = NC * NS
RPW = B // NW
G = 40
NCHUNK = L // G
DP = 304


def _sc_sums(x, table_pad):
  mesh = plsc.VectorSubcoreMesh(core_axis_name="c", subcore_axis_name="s")

  @functools.partial(
      pl.kernel,
      out_type=jax.ShapeDtypeStruct((B, DP), jnp.float32),
      mesh=mesh,
      compiler_params=pltpu.CompilerParams(
          use_tc_tiling_on_sc=False, needs_layout_passes=False),
      scratch_types=[
          pltpu.VMEM((G,), jnp.int32),
          pltpu.VMEM((G, DP), jnp.float32),
          pltpu.VMEM((DP,), jnp.float32),
          pltpu.SemaphoreType.DMA,
      ],
  )
  def k(x_hbm, tab_hbm, out_hbm, idxg_v, buf_v, acc_v, sem):
    wid = lax.axis_index("s") * NC + lax.axis_index("c")
    base = wid * RPW

    @pl.loop(0, RPW)
    def _row(i):
      for kk in range(0, DP, 16):
        acc_v[pl.ds(kk, 16)] = jnp.zeros((16,), jnp.float32)

      for c in range(NCHUNK):
        pltpu.sync_copy(x_hbm.at[base + i, pl.ds(c * G, G)], idxg_v)
        pltpu.async_copy(tab_hbm.at[idxg_v], buf_v, sem).wait()

        for k2 in range(DP // 16):
          off = k2 * 16

          def body(r, a, off=off):
            return a + buf_v[r, pl.ds(off, 16)]

          acc_v[pl.ds(off, 16)] = lax.fori_loop(
              0, G, body, acc_v[pl.ds(off, 16)])

      pltpu.sync_copy(acc_v, out_hbm.at[base + i])

  return k(x, table_pad)


def _mlp_body(s_ref, w1_ref, b1_ref, w2_ref, b2_ref, o_ref):
  h = jnp.dot(s_ref[...], w1_ref[...], preferred_element_type=jnp.float32)
  h = jnp.maximum(h + b1_ref[...], 0.0)
  logits = jnp.dot(h, w2_ref[...], preferred_element_type=jnp.float32)
  logits = logits + b2_ref[...]
  m = jnp.max(logits, axis=0, keepdims=True)
  z = logits - m
  o_ref[...] = z - jnp.log(jnp.sum(jnp.exp(z), axis=0, keepdims=True))


def kernel(x, emb_table, V_w, V_b, W_w, W_b):
  x = x.astype(jnp.int32)
  table_pad = jnp.pad(emb_table, ((0, 0), (0, DP - D)))
  sums = _sc_sums(x, table_pad)
  w1 = jnp.pad(V_w.T * (1.0 / L), ((0, DP - D), (0, 0)))
  b1 = V_b.reshape(1, HID)
  w2 = W_w.T
  b2 = W_b.reshape(1, OUT)
  return pl.pallas_call(
      _mlp_body,
      out_shape=jax.ShapeDtypeStruct((B, OUT), jnp.float32),
  )(sums, w1, b1, w2, b2)


# speedup vs baseline: 1.7453x; 1.7453x over previous
import functools

import jax
import jax.numpy as jnp
from jax import lax
from jax.experimental import pallas as pl
from jax.experimental.pallas import tpu as pltpu
from jax.experimental.pallas import tpu_sc as plsc

D = 300
B = 1024
L = 200
HID = 32
OUT = 2

NC = 2
NS = 16
NW = NC * NS
RPW = B // NW
G = 40
NCHUNK = L // G
DP = 304
NACC = DP // 16


def _sc_sums(x, table_pad):
  mesh = plsc.VectorSubcoreMesh(core_axis_name="c", subcore_axis_name="s")

  @functools.partial(
      pl.kernel,
      out_type=jax.ShapeDtypeStruct((B, DP), jnp.float32),
      mesh=mesh,
      compiler_params=pltpu.CompilerParams(
          use_tc_tiling_on_sc=False, needs_layout_passes=False),
      scratch_types=(
          [pltpu.VMEM((RPW, L), jnp.int32)]
          + [pltpu.VMEM((G, DP), jnp.float32) for _ in range(NCHUNK)]
          + [pltpu.VMEM((RPW, DP), jnp.float32)]
          + [pltpu.SemaphoreType.DMA for _ in range(NCHUNK)]
      ),
  )
  def k(x_hbm, tab_hbm, out_hbm, idx_v, *rest):
    bufs = rest[:NCHUNK]
    ostage = rest[NCHUNK]
    sems = rest[NCHUNK + 1:]
    wid = lax.axis_index("s") * NC + lax.axis_index("c")
    base = wid * RPW
    pltpu.sync_copy(x_hbm.at[pl.ds(base, RPW)], idx_v)

    for c in range(NCHUNK):
      pltpu.async_copy(tab_hbm.at[idx_v.at[0, pl.ds(c * G, G)]],
                       bufs[c], sems[c])

    @pl.loop(0, RPW)
    def _row(i):
      accs = tuple(jnp.zeros((16,), jnp.float32) for _ in range(NACC))
      for c in range(NCHUNK):
        pltpu.make_async_copy(tab_hbm.at[idx_v.at[i, pl.ds(c * G, G)]],
                              bufs[c], sems[c]).wait()

        def body(r, a, _buf=bufs[c]):
          return tuple(x + _buf[r, pl.ds(kk * 16, 16)]
                       for kk, x in enumerate(a))

        accs = lax.fori_loop(0, G, body, accs)

        @pl.when(i + 1 < RPW)
        def _():
          pltpu.async_copy(tab_hbm.at[idx_v.at[i + 1, pl.ds(c * G, G)]],
                           bufs[c], sems[c])

      for kk in range(NACC):
        ostage[i, pl.ds(kk * 16, 16)] = accs[kk]

    pltpu.sync_copy(ostage, out_hbm.at[pl.ds(base, RPW)])

  return k(x, table_pad)


def _mlp_body(s_ref, w1_ref, b1_ref, w2_ref, b2_ref, o_ref):
  h = jnp.dot(s_ref[...], w1_ref[...], preferred_element_type=jnp.float32)
  h = jnp.maximum(h + b1_ref[...], 0.0)
  logits = jnp.dot(h, w2_ref[...], preferred_element_type=jnp.float32)
  logits = logits + b2_ref[...]
  m = jnp.max(logits, axis=0, keepdims=True)
  z = logits - m
  o_ref[...] = z - jnp.log(jnp.sum(jnp.exp(z), axis=0, keepdims=True))


def kernel(x, emb_table, V_w, V_b, W_w, W_b):
  x = x.astype(jnp.int32)
  table_pad = jnp.pad(emb_table, ((0, 0), (0, DP - D)))
  sums = _sc_sums(x, table_pad)
  w1 = jnp.pad(V_w.T * (1.0 / L), ((0, DP - D), (0, 0)))
  b1 = V_b.reshape(1, HID)
  w2 = W_w.T
  b2 = W_b.reshape(1, OUT)
  return pl.pallas_call(
      _mlp_body,
      out_shape=jax.ShapeDtypeStruct((B, OUT), jnp.float32),
  )(sums, w1, b1, w2, b2)


# speedup vs baseline: 3.0207x; 1.7307x over previous
import functools

import jax
import jax.numpy as jnp
from jax import lax
from jax.experimental import pallas as pl
from jax.experimental.pallas import tpu as pltpu
from jax.experimental.pallas import tpu_sc as plsc

VOCAB = 100000
D = 300
B = 1024
L = 200
HID = 32
OUT = 2

NC = 2
NS = 16
NW = NC * NS
RPW = B // NW
G = 40
NCHUNK = L // G
DP = 304
NACC = DP // 16


def _sc_sums(x, table_pad):
  mesh = plsc.VectorSubcoreMesh(core_axis_name="c", subcore_axis_name="s")

  @functools.partial(
      pl.kernel,
      out_type=jax.ShapeDtypeStruct((B, DP), jnp.float32),
      mesh=mesh,
      compiler_params=pltpu.CompilerParams(
          use_tc_tiling_on_sc=False, needs_layout_passes=False),
      scratch_types=(
          [pltpu.VMEM((RPW, L), jnp.int32)]
          + [pltpu.VMEM((G, DP), jnp.float32) for _ in range(NCHUNK)]
          + [pltpu.VMEM((RPW, DP), jnp.float32)]
          + [pltpu.SemaphoreType.DMA for _ in range(NCHUNK)]
      ),
  )
  def k(x_hbm, tab_hbm, out_hbm, idx_v, *rest):
    bufs = rest[:NCHUNK]
    ostage = rest[NCHUNK]
    sems = rest[NCHUNK + 1:]
    wid = lax.axis_index("s") * NC + lax.axis_index("c")
    base = wid * RPW
    pltpu.sync_copy(x_hbm.at[pl.ds(base, RPW)], idx_v)

    for c in range(NCHUNK):
      pltpu.async_copy(tab_hbm.at[idx_v.at[0, pl.ds(c * G, G)]],
                       bufs[c], sems[c])

    @pl.loop(0, RPW)
    def _row(i):
      accs = tuple(jnp.zeros((16,), jnp.float32) for _ in range(NACC))
      for c in range(NCHUNK):
        pltpu.make_async_copy(tab_hbm.at[idx_v.at[i, pl.ds(c * G, G)]],
                              bufs[c], sems[c]).wait()

        def body(r, a, _buf=bufs[c]):
          return tuple(x + _buf[r, pl.ds(kk * 16, 16)]
                       for kk, x in enumerate(a))

        accs = lax.fori_loop(0, G, body, accs)

        @pl.when(i + 1 < RPW)
        def _():
          pltpu.async_copy(tab_hbm.at[idx_v.at[i + 1, pl.ds(c * G, G)]],
                           bufs[c], sems[c])

      for kk in range(NACC):
        ostage[i, pl.ds(kk * 16, 16)] = accs[kk]

    pltpu.sync_copy(ostage, out_hbm.at[pl.ds(base, RPW)])

  return k(x, table_pad)


PAD_ROWS = 4000


def _pad_body(t_ref, o_ref):
  o_ref[...] = jnp.concatenate(
      [t_ref[...], jnp.zeros((PAD_ROWS, DP - D), jnp.float32)], axis=1)


def _pad_table(emb_table):
  return pl.pallas_call(
      _pad_body,
      grid=(VOCAB // PAD_ROWS,),
      in_specs=[pl.BlockSpec((PAD_ROWS, D), lambda i: (i, 0))],
      out_specs=pl.BlockSpec((PAD_ROWS, DP), lambda i: (i, 0)),
      out_shape=jax.ShapeDtypeStruct((VOCAB, DP), jnp.float32),
  )(emb_table)


def _mlp_body(s_ref, w1_ref, b1_ref, w2_ref, b2_ref, o_ref):
  h = jnp.dot(s_ref[...], w1_ref[...], preferred_element_type=jnp.float32)
  h = jnp.maximum(h + b1_ref[...], 0.0)
  logits = jnp.dot(h, w2_ref[...], preferred_element_type=jnp.float32)
  logits = logits + b2_ref[...]
  m = jnp.max(logits, axis=0, keepdims=True)
  z = logits - m
  o_ref[...] = z - jnp.log(jnp.sum(jnp.exp(z), axis=0, keepdims=True))


def kernel(x, emb_table, V_w, V_b, W_w, W_b):
  x = x.astype(jnp.int32)
  table_pad = _pad_table(emb_table)
  sums = _sc_sums(x, table_pad)
  w1 = jnp.pad(V_w.T * (1.0 / L), ((0, DP - D), (0, 0)))
  b1 = V_b.reshape(1, HID)
  w2 = W_w.T
  b2 = W_b.reshape(1, OUT)
  return pl.pallas_call(
      _mlp_body,
      out_shape=jax.ShapeDtypeStruct((B, OUT), jnp.float32),
  )(sums, w1, b1, w2, b2)


# speedup vs baseline: 3.0238x; 1.0010x over previous
import functools

import jax
import jax.numpy as jnp
from jax import lax
from jax.experimental import pallas as pl
from jax.experimental.pallas import tpu as pltpu
from jax.experimental.pallas import tpu_sc as plsc

VOCAB = 100000
D = 300
B = 1024
L = 200
HID = 32
OUT = 2

NC = 2
NS = 16
NW = NC * NS
RPW = B // NW
G = 40
NCHUNK = L // G
DP = 304
NACC = DP // 16


def _sc_sums(x, table_pad):
  mesh = plsc.VectorSubcoreMesh(core_axis_name="c", subcore_axis_name="s")

  @functools.partial(
      pl.kernel,
      out_type=jax.ShapeDtypeStruct((B, DP), jnp.float32),
      mesh=mesh,
      compiler_params=pltpu.CompilerParams(
          use_tc_tiling_on_sc=False, needs_layout_passes=False),
      scratch_types=(
          [pltpu.VMEM((RPW, L), jnp.int32)]
          + [pltpu.VMEM((G, DP), jnp.float32) for _ in range(NCHUNK)]
          + [pltpu.VMEM((RPW, DP), jnp.float32)]
          + [pltpu.SemaphoreType.DMA for _ in range(NCHUNK)]
      ),
  )
  def k(x_hbm, tab_hbm, out_hbm, idx_v, *rest):
    bufs = rest[:NCHUNK]
    ostage = rest[NCHUNK]
    sems = rest[NCHUNK + 1:]
    wid = lax.axis_index("s") * NC + lax.axis_index("c")
    base = wid * RPW
    pltpu.sync_copy(x_hbm.at[pl.ds(base, RPW)], idx_v)

    for c in range(NCHUNK):
      pltpu.async_copy(tab_hbm.at[idx_v.at[0, pl.ds(c * G, G)]],
                       bufs[c], sems[c])

    @pl.loop(0, RPW)
    def _row(i):
      accs = tuple(jnp.zeros((16,), jnp.float32) for _ in range(NACC))
      for c in range(NCHUNK):
        pltpu.make_async_copy(tab_hbm.at[idx_v.at[i, pl.ds(c * G, G)]],
                              bufs[c], sems[c]).wait()

        def body(r, a, _buf=bufs[c]):
          return tuple(x + _buf[r, pl.ds(kk * 16, 16)]
                       for kk, x in enumerate(a))

        accs = lax.fori_loop(0, G, body, accs)

        @pl.when(i + 1 < RPW)
        def _():
          pltpu.async_copy(tab_hbm.at[idx_v.at[i + 1, pl.ds(c * G, G)]],
                           bufs[c], sems[c])

      for kk in range(NACC):
        ostage[i, pl.ds(kk * 16, 16)] = accs[kk]

    pltpu.sync_copy(ostage, out_hbm.at[pl.ds(base, RPW)])

  return k(x, table_pad)


PAD_ROWS = 10000


def _pad_body(t_ref, o_ref):
  o_ref[:, pl.ds(0, D)] = t_ref[...]


def _pad_table(emb_table):
  return pl.pallas_call(
      _pad_body,
      grid=(VOCAB // PAD_ROWS,),
      in_specs=[pl.BlockSpec((PAD_ROWS, D), lambda i: (i, 0))],
      out_specs=pl.BlockSpec((PAD_ROWS, DP), lambda i: (i, 0)),
      out_shape=jax.ShapeDtypeStruct((VOCAB, DP), jnp.float32),
  )(emb_table)


def _mlp_body(s_ref, w1_ref, b1_ref, w2_ref, b2_ref, o_ref):
  h = jnp.dot(s_ref[:, pl.ds(0, D)], w1_ref[...],
              preferred_element_type=jnp.float32)
  h = jnp.maximum(h + b1_ref[...], 0.0)
  logits = jnp.dot(h, w2_ref[...], preferred_element_type=jnp.float32)
  logits = logits + b2_ref[...]
  m = jnp.max(logits, axis=0, keepdims=True)
  z = logits - m
  o_ref[...] = z - jnp.log(jnp.sum(jnp.exp(z), axis=0, keepdims=True))


def kernel(x, emb_table, V_w, V_b, W_w, W_b):
  x = x.astype(jnp.int32)
  table_pad = _pad_table(emb_table)
  sums = _sc_sums(x, table_pad)
  w1 = V_w.T * (1.0 / L)
  b1 = V_b.reshape(1, HID)
  w2 = W_w.T
  b2 = W_b.reshape(1, OUT)
  return pl.pallas_call(
      _mlp_body,
      out_shape=jax.ShapeDtypeStruct((B, OUT), jnp.float32),
  )(sums, w1, b1, w2, b2)
